# baseline (speedup 1.0000x reference)
import functools

import jax
import jax.numpy as jnp
from jax import lax
from jax.experimental import pallas as pl
from jax.experimental.pallas import tpu as pltpu
from jax.experimental.pallas import tpu_sc as plsc

NU = 25000
NI = 25000
E = 400000
D = 64
HW = 32

NC = 2
NS = 16
NW = NC * NS
NPAD = 25088
STRIPE = NPAD // NS
EPAD = 401408
TCH = EPAD // NW
CHK = 128
NCHK = TCH // CHK
NBUF = 4
TRASH = NPAD - 1

_mesh = functools.partial(
    plsc.VectorSubcoreMesh, core_axis_name="c", subcore_axis_name="s",
    num_cores=NC, num_subcores=NS)

_params = pltpu.CompilerParams(use_tc_tiling_on_sc=False,
                               needs_layout_passes=False)


def _fill_rows(rows, width, value):
    nv = width // 16

    def zb(i, carry):
        rows[i // nv, pl.ds((i % nv) * 16, 16)] = jnp.full((16,), value, jnp.float32)
        return carry
    lax.fori_loop(0, CHK * nv, zb, 0)


def _zero_acc_stripe(zrows, acc, base):
    for t in range(STRIPE // 112):
        pltpu.sync_copy(zrows.at[pl.ds(0, 112)],
                        acc.at[pl.ds(base + t * 112, 112)])


def _deg_pass(dst2):
    W = 16

    @functools.partial(
        pl.kernel,
        out_type=jax.ShapeDtypeStruct((NC, NPAD, W), jnp.float32),
        mesh=_mesh(),
        compiler_params=_params,
        scratch_types=[
            pltpu.VMEM((NCHK, CHK), jnp.int32),
            pltpu.VMEM((CHK, W), jnp.float32),
            pltpu.VMEM((CHK, W), jnp.float32),
            pltpu.VMEM_SHARED((NPAD, W), jnp.float32),
        ],
    )
    def k(didx_ref, out_ref, idx_d, zrows, orows, acc):
        c = lax.axis_index("c")
        s = lax.axis_index("s")
        wid = c * NS + s
        _fill_rows(zrows, W, 0.0)
        _fill_rows(orows, W, 1.0)
        base = s * STRIPE
        _zero_acc_stripe(zrows, acc, base)
        plsc.subcore_barrier()
        pltpu.sync_copy(didx_ref.at[pl.ds(wid * NCHK, NCHK)], idx_d)

        def eb(kk, carry):
            pltpu.sync_copy(orows, acc.at[idx_d.at[kk]], add=True)
            return carry
        lax.fori_loop(0, NCHK, eb, 0)
        plsc.subcore_barrier()
        pltpu.sync_copy(acc.at[pl.ds(base, STRIPE)],
                        out_ref.at[c, pl.ds(base, STRIPE)])
    return k(dst2)


def _gather_scalar(table, idx2):
    @functools.partial(
        pl.kernel,
        out_type=jax.ShapeDtypeStruct((EPAD,), jnp.float32),
        mesh=_mesh(),
        compiler_params=_params,
        scratch_types=[
            pltpu.VMEM((NPAD,), jnp.float32),
            pltpu.VMEM((NCHK, CHK), jnp.int32),
            pltpu.VMEM((TCH,), jnp.float32),
        ],
    )
    def k(table_ref, idx_ref, out_ref, tab_v, idx_v, s_v):
        c = lax.axis_index("c")
        s = lax.axis_index("s")
        wid = c * NS + s
        pltpu.sync_copy(table_ref, tab_v)
        pltpu.sync_copy(idx_ref.at[pl.ds(wid * NCHK, NCHK)], idx_v)
        nv = CHK // 16

        def gb(j, carry):
            idx16 = idx_v[j // nv, pl.ds((j % nv) * 16, 16)]
            s_v[pl.ds(j * 16, 16)] = plsc.load_gather(tab_v, [idx16])
            return carry
        lax.fori_loop(0, TCH // 16, gb, 0)
        pltpu.sync_copy(s_v, out_ref.at[pl.ds(wid * TCH, TCH)])
    return k(table, idx2)


def _edge_pass(table, src2, dst2, tok, linear=False, src_limit=None):
    scratch = [
        pltpu.VMEM((NCHK, CHK), jnp.int32),
        pltpu.VMEM((16,), jnp.float32),
        pltpu.VMEM_SHARED((NPAD, HW), jnp.float32),
    ]
    if not linear:
        scratch.append(pltpu.VMEM((NCHK, CHK), jnp.int32))
    scratch += [pltpu.VMEM((CHK, HW), jnp.float32) for _ in range(NBUF)]
    scratch += [pltpu.SemaphoreType.DMA for _ in range(NBUF)]

    @functools.partial(
        pl.kernel,
        out_type=jax.ShapeDtypeStruct((NC, NPAD, HW), jnp.float32),
        mesh=_mesh(),
        compiler_params=_params,
        scratch_types=scratch,
    )
    def k(*refs):
        if linear:
            (table_ref, didx_ref, tok_ref, out_ref, idx_d, tok_v, acc), rest = \
                refs[:7], refs[7:]
            idx_s = None
        else:
            (table_ref, sidx_ref, didx_ref, tok_ref, out_ref, idx_d, tok_v,
             acc, idx_s), rest = refs[:9], refs[9:]
        bufs, sems = rest[:NBUF], rest[NBUF:]
        c = lax.axis_index("c")
        s = lax.axis_index("s")
        wid = c * NS + s
        pltpu.sync_copy(tok_ref, tok_v)
        _fill_rows(bufs[0], HW, 0.0)
        base = s * STRIPE
        _zero_acc_stripe(bufs[0], acc, base)
        plsc.subcore_barrier()
        pltpu.sync_copy(didx_ref.at[pl.ds(wid * NCHK, NCHK)], idx_d)
        if not linear:
            pltpu.sync_copy(sidx_ref.at[pl.ds(wid * NCHK, NCHK)], idx_s)
        eb0 = wid * TCH

        def src_ref(kk):
            if linear:
                return table_ref.at[pl.ds(eb0 + kk * CHK, CHK)]
            return table_ref.at[idx_s.at[kk]]

        def valid(kk):
            if linear and src_limit is not None:
                return eb0 + kk * CHK < src_limit
            return kk == kk

        def issue(kk, buf, sem):
            if linear and src_limit is not None:
                @pl.when(eb0 + kk * CHK < src_limit)
                def _():
                    pltpu.async_copy(src_ref(kk), buf, sem)
            else:
                pltpu.async_copy(src_ref(kk), buf, sem)

        def consume(kk, buf, sem):
            if linear and src_limit is not None:
                @pl.when(eb0 + kk * CHK < src_limit)
                def _():
                    pltpu.make_async_copy(src_ref(kk), buf, sem).wait()
                    pltpu.sync_copy(buf, acc.at[idx_d.at[kk]], add=True)
            else:
                pltpu.make_async_copy(src_ref(kk), buf, sem).wait()
                pltpu.sync_copy(buf, acc.at[idx_d.at[kk]], add=True)

        for p in range(NBUF - 1):
            issue(p, bufs[p], sems[p])

        NFULL = (NCHK // NBUF) * NBUF

        def gb(g, carry):
            for j in range(NBUF):
                kk = g * NBUF + j

                @pl.when(kk + NBUF - 1 < NCHK)
                def _():
                    issue(kk + NBUF - 1, bufs[(j + NBUF - 1) % NBUF],
                          sems[(j + NBUF - 1) % NBUF])
                consume(kk, bufs[j], sems[j])
            return carry
        lax.fori_loop(0, NFULL // NBUF, gb, 0)
        for kk in range(NFULL, NCHK):
            consume(kk, bufs[kk % NBUF], sems[kk % NBUF])
        plsc.subcore_barrier()
        pltpu.sync_copy(acc.at[pl.ds(base, STRIPE)],
                        out_ref.at[c, pl.ds(base, STRIPE)])

    if linear:
        return k(table, dst2, tok)
    return k(table, src2, dst2, tok)


def _gather_rows(table_a, table_b, idx_a2, idx_b2, batch):
    per_w = batch // NW
    nchk = per_w // CHK

    @functools.partial(
        pl.kernel,
        out_type=(jax.ShapeDtypeStruct((batch, D), jnp.float32),
                  jax.ShapeDtypeStruct((batch, D), jnp.float32)),
        mesh=_mesh(),
        compiler_params=_params,
        scratch_types=[
            pltpu.VMEM((nchk, CHK), jnp.int32),
            pltpu.VMEM((nchk, CHK), jnp.int32),
            pltpu.VMEM((CHK, D), jnp.float32),
            pltpu.VMEM((CHK, D), jnp.float32),
            pltpu.SemaphoreType.DMA,
            pltpu.SemaphoreType.DMA,
        ],
    )
    def k(ta_ref, tb_ref, ia_ref, ib_ref, oa_ref, ob_ref, ia, ib,
          rows0, rows1, sem0, sem1):
        c = lax.axis_index("c")
        s = lax.axis_index("s")
        wid = c * NS + s
        pltpu.sync_copy(ia_ref.at[pl.ds(wid * nchk, nchk)], ia)
        pltpu.sync_copy(ib_ref.at[pl.ds(wid * nchk, nchk)], ib)
        pltpu.async_copy(ta_ref.at[ia.at[0]], rows0, sem0)
        pltpu.async_copy(tb_ref.at[ib.at[0]], rows1, sem1)

        def eb(kk, carry):
            off = wid * per_w + kk * CHK
            pltpu.make_async_copy(ta_ref.at[ia.at[kk]], rows0, sem0).wait()
            pltpu.sync_copy(rows0, oa_ref.at[pl.ds(off, CHK)])

            @pl.when(kk + 1 < nchk)
            def _():
                pltpu.async_copy(ta_ref.at[ia.at[kk + 1]], rows0, sem0)
            pltpu.make_async_copy(tb_ref.at[ib.at[kk]], rows1, sem1).wait()
            pltpu.sync_copy(rows1, ob_ref.at[pl.ds(off, CHK)])

            @pl.when(kk + 1 < nchk)
            def _():
                pltpu.async_copy(tb_ref.at[ib.at[kk + 1]], rows1, sem1)
            return carry
        lax.fori_loop(0, nchk, eb, 0)
    return k(table_a, table_b, idx_a2, idx_b2)


def _mm_ee_body(a_ref, f_ref, s_ref, lo_ref, hi_ref):
    r = jnp.dot(a_ref[...], f_ref[...], preferred_element_type=jnp.float32)
    r = r * (0.7 * s_ref[...])
    lo_ref[...] = r[:, :HW]
    hi_ref[...] = r[:, HW:]


def _mm_ee(ef, F, s):
    blk = 2000
    grid = (E // blk,)
    return pl.pallas_call(
        _mm_ee_body,
        out_shape=(jax.ShapeDtypeStruct((E, HW), jnp.float32),
                   jax.ShapeDtypeStruct((E, HW), jnp.float32)),
        grid=grid,
        in_specs=[
            pl.BlockSpec((blk, D), lambda i: (i, 0)),
            pl.BlockSpec((D, D), lambda i: (0, 0)),
            pl.BlockSpec((blk, 1), lambda i: (i, 0)),
        ],
        out_specs=(pl.BlockSpec((blk, HW), lambda i: (i, 0)),
                   pl.BlockSpec((blk, HW), lambda i: (i, 0))),
    )(ef, F, s)


def _mm_if_body(a_ref, f_ref, lo_ref, hi_ref):
    r = jnp.dot(a_ref[...], f_ref[...], preferred_element_type=jnp.float32)
    r = r * 0.8
    lo_ref[...] = r[:, :HW]
    hi_ref[...] = r[:, HW:]


def _mm_if(ifeat_pad, F):
    blk = 1568
    return pl.pallas_call(
        _mm_if_body,
        out_shape=(jax.ShapeDtypeStruct((NPAD, HW), jnp.float32),
                   jax.ShapeDtypeStruct((NPAD, HW), jnp.float32)),
        grid=(NPAD // blk,),
        in_specs=[
            pl.BlockSpec((blk, D), lambda i: (i, 0)),
            pl.BlockSpec((D, D), lambda i: (0, 0)),
        ],
        out_specs=(pl.BlockSpec((blk, HW), lambda i: (i, 0)),
                   pl.BlockSpec((blk, HW), lambda i: (i, 0))),
    )(ifeat_pad, F)


_UBLK = 1568


def _nspec():
    return pl.BlockSpec((_UBLK, HW), lambda i: (i, 0))


def _pspec():
    return pl.BlockSpec((NC, _UBLK, HW), lambda i: (0, i, 0))


def _init_body(cil, cih, cul, cuh, gu, gi, dv,
               Cil, Cih, Cul, Cuh, yul, yuh, yil, yih):
    Cil[...] = cil[0] + cil[1]
    Cih[...] = cih[0] + cih[1]
    Cul[...] = cul[0] + cul[1]
    Cuh[...] = cuh[0] + cuh[1]
    sc = 0.3 * dv[...]
    yul[...] = sc * gu[:, :HW]
    yuh[...] = sc * gu[:, HW:]
    yil[...] = 0.2 * gi[:, :HW]
    yih[...] = 0.2 * gi[:, HW:]


def _init_pass(ci_lo, ci_hi, cu_lo, cu_hi, gu_pad, gi_pad, dinv2):
    outs = tuple(jax.ShapeDtypeStruct((NPAD, HW), jnp.float32) for _ in range(8))
    return pl.pallas_call(
        _init_body,
        out_shape=outs,
        grid=(NPAD // _UBLK,),
        in_specs=[
            _pspec(), _pspec(), _pspec(), _pspec(),
            pl.BlockSpec((_UBLK, D), lambda i: (i, 0)),
            pl.BlockSpec((_UBLK, D), lambda i: (i, 0)),
            pl.BlockSpec((_UBLK, 1), lambda i: (i, 0)),
        ],
        out_specs=tuple(_nspec() for _ in range(8)),
    )(ci_lo, ci_hi, cu_lo, cu_hi, gu_pad, gi_pad, dinv2)


def _make_update_body(alpha, want_y):
    def body(pil, pih, pul, puh, Cil, Cih, Cul, Cuh, dv,
             aul_i, auh_i, ail_i, aih_i, *outs):
        xil = Cil[...] + pil[0] + pil[1]
        xih = Cih[...] + pih[0] + pih[1]
        xul = Cul[...] + pul[0] + pul[1]
        xuh = Cuh[...] + puh[0] + puh[1]
        if want_y:
            (aul, auh, ail, aih, yul, yuh, yil, yih) = outs
            sc = 0.3 * dv[...]
            yul[...] = sc * xul
            yuh[...] = sc * xuh
            yil[...] = 0.2 * xil
            yih[...] = 0.2 * xih
        else:
            (aul, auh, ail, aih) = outs
        aul[...] = aul_i[...] + xul * alpha
        auh[...] = auh_i[...] + xuh * alpha
        ail[...] = ail_i[...] + xil * alpha
        aih[...] = aih_i[...] + xih * alpha
    return body


def _update_pass(pi_lo, pi_hi, pu_lo, pu_hi, Cil, Cih, Cul, Cuh, dinv2,
                 aul, auh, ail, aih, alpha, want_y):
    n_out = 8 if want_y else 4
    outs = tuple(jax.ShapeDtypeStruct((NPAD, HW), jnp.float32)
                 for _ in range(n_out))
    return pl.pallas_call(
        _make_update_body(alpha, want_y),
        out_shape=outs,
        grid=(NPAD // _UBLK,),
        in_specs=[
            _pspec(), _pspec(), _pspec(), _pspec(),
            _nspec(), _nspec(), _nspec(), _nspec(),
            pl.BlockSpec((_UBLK, 1), lambda i: (i, 0)),
            _nspec(), _nspec(), _nspec(), _nspec(),
        ],
        out_specs=tuple(_nspec() for _ in range(n_out)),
    )(pi_lo, pi_hi, pu_lo, pu_hi, Cil, Cih, Cul, Cuh, dinv2,
      aul, auh, ail, aih)


def _dot_body(a_ref, b_ref, o_ref):
    o_ref[...] = jnp.sum(a_ref[...] * b_ref[...], axis=1, keepdims=True)


def _batched_dot(a, b):
    B, Dd = a.shape
    blk = 1024
    return pl.pallas_call(
        _dot_body,
        out_shape=jax.ShapeDtypeStruct((B, 1), jnp.float32),
        grid=(B // blk,),
        in_specs=[
            pl.BlockSpec((blk, Dd), lambda i: (i, 0)),
            pl.BlockSpec((blk, Dd), lambda i: (i, 0)),
        ],
        out_specs=pl.BlockSpec((blk, 1), lambda i: (i, 0)),
    )(a, b)


def _pad_rows(x, n):
    return jnp.zeros((n, x.shape[1]), x.dtype).at[: x.shape[0]].set(x)


def kernel(Gu, Gi, F, edge_features, item_features, edge_index, user_idx, item_idx):
    u = edge_index[0, :E]
    items = edge_index[1, :E] - NU

    u_src = jnp.full((EPAD,), NU, jnp.int32).at[:E].set(u).reshape(-1, CHK)
    it_src = jnp.full((EPAD,), NI, jnp.int32).at[:E].set(items).reshape(-1, CHK)
    u_dst = jnp.full((EPAD,), TRASH, jnp.int32).at[:E].set(u).reshape(-1, CHK)
    it_dst = jnp.full((EPAD,), TRASH, jnp.int32).at[:E].set(items).reshape(-1, CHK)

    degp = _deg_pass(u_dst)
    deg = degp[0, :, 0] + degp[1, :, 0]
    dinv_pad = jnp.where(deg > 0, 1.0 / deg, 0.0)
    dinv2 = dinv_pad[:, None]

    s = _gather_scalar(dinv_pad, u_src)
    ee_lo, ee_hi = _mm_ee(edge_features, F, s[:E, None])
    if_lo, if_hi = _mm_if(_pad_rows(item_features, NPAD), F)

    ci_lo = _edge_pass(ee_lo, None, it_dst, s[:16], linear=True, src_limit=E)
    ci_hi = _edge_pass(ee_hi, None, it_dst, ci_lo[0, 0, :16], linear=True,
                       src_limit=E)
    cu_lo = _edge_pass(if_lo, it_src, u_dst, ci_hi[0, 0, :16])
    cu_hi = _edge_pass(if_hi, it_src, u_dst, cu_lo[0, 0, :16])

    gu_pad = _pad_rows(Gu, NPAD)
    gi_pad = _pad_rows(Gi, NPAD)
    (Cil, Cih, Cul, Cuh, yul, yuh, yil, yih) = _init_pass(
        ci_lo, ci_hi, cu_lo, cu_hi, gu_pad, gi_pad, dinv2)
    aul, auh = gu_pad[:, :HW], gu_pad[:, HW:]
    ail, aih = gi_pad[:, :HW], gi_pad[:, HW:]

    for layer in range(3):
        pi_lo = _edge_pass(yul, u_src, it_dst, yih[0, :16])
        pi_hi = _edge_pass(yuh, u_src, it_dst, pi_lo[0, 0, :16])
        pu_lo = _edge_pass(yil, it_src, u_dst, pi_hi[0, 0, :16])
        pu_hi = _edge_pass(yih, it_src, u_dst, pu_lo[0, 0, :16])
        alpha = 1.0 / (layer + 2)
        want_y = layer < 2
        res = _update_pass(pi_lo, pi_hi, pu_lo, pu_hi, Cil, Cih, Cul, Cuh,
                           dinv2, aul, auh, ail, aih, alpha, want_y)
        if want_y:
            (aul, auh, ail, aih, yul, yuh, yil, yih) = res
        else:
            (aul, auh, ail, aih) = res

    au = jnp.concatenate([aul, auh], axis=1)
    ai = jnp.concatenate([ail, aih], axis=1)
    ui2 = user_idx.astype(jnp.int32).reshape(-1, CHK)
    ii2 = item_idx.astype(jnp.int32).reshape(-1, CHK)
    ga, gb = _gather_rows(au, ai, ui2, ii2, user_idx.shape[0])
    return _batched_dot(ga, gb)[:, 0]


# speedup vs baseline: 1.0234x; 1.0234x over previous
import functools

import jax
import jax.numpy as jnp
from jax import lax
from jax.experimental import pallas as pl
from jax.experimental.pallas import tpu as pltpu
from jax.experimental.pallas import tpu_sc as plsc

NU = 25000
NI = 25000
E = 400000
D = 64
HW = 32

NC = 2
NS = 16
NW = NC * NS
NPAD = 25088
STRIPE = NPAD // NS
EPAD = 401408
TCH = EPAD // NW
CHK = 128
NCHK = TCH // CHK
NBUF = 4
TRASH = NPAD - 1

_mesh = functools.partial(
    plsc.VectorSubcoreMesh, core_axis_name="c", subcore_axis_name="s",
    num_cores=NC, num_subcores=NS)

_params = pltpu.CompilerParams(use_tc_tiling_on_sc=False,
                               needs_layout_passes=False)


def _fill_rows(rows, width, value):
    nv = width // 16

    def zb(i, carry):
        rows[i // nv, pl.ds((i % nv) * 16, 16)] = jnp.full((16,), value, jnp.float32)
        return carry
    lax.fori_loop(0, CHK * nv, zb, 0)


def _zero_acc_stripe(zrows, acc, base):
    for t in range(STRIPE // 112):
        pltpu.sync_copy(zrows.at[pl.ds(0, 112)],
                        acc.at[pl.ds(base + t * 112, 112)])


def _deg_pass(dst2):
    W = 16

    @functools.partial(
        pl.kernel,
        out_type=jax.ShapeDtypeStruct((NC, NPAD, W), jnp.float32),
        mesh=_mesh(),
        compiler_params=_params,
        scratch_types=[
            pltpu.VMEM((NCHK, CHK), jnp.int32),
            pltpu.VMEM((CHK, W), jnp.float32),
            pltpu.VMEM((CHK, W), jnp.float32),
            pltpu.VMEM_SHARED((NPAD, W), jnp.float32),
        ],
    )
    def k(didx_ref, out_ref, idx_d, zrows, orows, acc):
        c = lax.axis_index("c")
        s = lax.axis_index("s")
        wid = c * NS + s
        _fill_rows(zrows, W, 0.0)
        _fill_rows(orows, W, 1.0)
        base = s * STRIPE
        _zero_acc_stripe(zrows, acc, base)
        plsc.subcore_barrier()
        pltpu.sync_copy(didx_ref.at[pl.ds(wid * NCHK, NCHK)], idx_d)

        def eb(kk, carry):
            pltpu.sync_copy(orows, acc.at[idx_d.at[kk]], add=True)
            return carry
        lax.fori_loop(0, NCHK, eb, 0)
        plsc.subcore_barrier()
        pltpu.sync_copy(acc.at[pl.ds(base, STRIPE)],
                        out_ref.at[c, pl.ds(base, STRIPE)])
    return k(dst2)


def _gather_scalar(table, idx2):
    @functools.partial(
        pl.kernel,
        out_type=jax.ShapeDtypeStruct((EPAD,), jnp.float32),
        mesh=_mesh(),
        compiler_params=_params,
        scratch_types=[
            pltpu.VMEM((NPAD,), jnp.float32),
            pltpu.VMEM((NCHK, CHK), jnp.int32),
            pltpu.VMEM((TCH,), jnp.float32),
        ],
    )
    def k(table_ref, idx_ref, out_ref, tab_v, idx_v, s_v):
        c = lax.axis_index("c")
        s = lax.axis_index("s")
        wid = c * NS + s
        pltpu.sync_copy(table_ref, tab_v)
        pltpu.sync_copy(idx_ref.at[pl.ds(wid * NCHK, NCHK)], idx_v)
        nv = CHK // 16

        def gb(r, carry):
            for q in range(nv):
                idx16 = idx_v[r, pl.ds(q * 16, 16)]
                s_v[pl.ds(r * CHK + q * 16, 16)] = \
                    plsc.load_gather(tab_v, [idx16])
            return carry
        lax.fori_loop(0, NCHK, gb, 0)
        pltpu.sync_copy(s_v, out_ref.at[pl.ds(wid * TCH, TCH)])
    return k(table, idx2)


def _edge_pass(table, src2, dst2, tok, linear=False, src_limit=None):
    scratch = [
        pltpu.VMEM((NCHK, CHK), jnp.int32),
        pltpu.VMEM((16,), jnp.float32),
        pltpu.VMEM_SHARED((NPAD, HW), jnp.float32),
    ]
    if not linear:
        scratch.append(pltpu.VMEM((NCHK, CHK), jnp.int32))
    scratch += [pltpu.VMEM((CHK, HW), jnp.float32) for _ in range(NBUF)]
    scratch += [pltpu.SemaphoreType.DMA for _ in range(NBUF)]

    @functools.partial(
        pl.kernel,
        out_type=jax.ShapeDtypeStruct((NC, NPAD, HW), jnp.float32),
        mesh=_mesh(),
        compiler_params=_params,
        scratch_types=scratch,
    )
    def k(*refs):
        if linear:
            (table_ref, didx_ref, tok_ref, out_ref, idx_d, tok_v, acc), rest = \
                refs[:7], refs[7:]
            idx_s = None
        else:
            (table_ref, sidx_ref, didx_ref, tok_ref, out_ref, idx_d, tok_v,
             acc, idx_s), rest = refs[:9], refs[9:]
        bufs, sems = rest[:NBUF], rest[NBUF:]
        c = lax.axis_index("c")
        s = lax.axis_index("s")
        wid = c * NS + s
        pltpu.sync_copy(tok_ref, tok_v)
        _fill_rows(bufs[0], HW, 0.0)
        base = s * STRIPE
        _zero_acc_stripe(bufs[0], acc, base)
        plsc.subcore_barrier()
        pltpu.sync_copy(didx_ref.at[pl.ds(wid * NCHK, NCHK)], idx_d)
        if not linear:
            pltpu.sync_copy(sidx_ref.at[pl.ds(wid * NCHK, NCHK)], idx_s)
        eb0 = wid * TCH

        def src_ref(kk):
            if linear:
                return table_ref.at[pl.ds(eb0 + kk * CHK, CHK)]
            return table_ref.at[idx_s.at[kk]]

        def valid(kk):
            if linear and src_limit is not None:
                return eb0 + kk * CHK < src_limit
            return kk == kk

        def issue(kk, buf, sem):
            if linear and src_limit is not None:
                @pl.when(eb0 + kk * CHK < src_limit)
                def _():
                    pltpu.async_copy(src_ref(kk), buf, sem)
            else:
                pltpu.async_copy(src_ref(kk), buf, sem)

        def consume(kk, buf, sem):
            if linear and src_limit is not None:
                @pl.when(eb0 + kk * CHK < src_limit)
                def _():
                    pltpu.make_async_copy(src_ref(kk), buf, sem).wait()
                    pltpu.sync_copy(buf, acc.at[idx_d.at[kk]], add=True)
            else:
                pltpu.make_async_copy(src_ref(kk), buf, sem).wait()
                pltpu.sync_copy(buf, acc.at[idx_d.at[kk]], add=True)

        for p in range(NBUF - 1):
            issue(p, bufs[p], sems[p])

        NFULL = (NCHK // NBUF) * NBUF

        def gb(g, carry):
            for j in range(NBUF):
                kk = g * NBUF + j

                @pl.when(kk + NBUF - 1 < NCHK)
                def _():
                    issue(kk + NBUF - 1, bufs[(j + NBUF - 1) % NBUF],
                          sems[(j + NBUF - 1) % NBUF])
                consume(kk, bufs[j], sems[j])
            return carry
        lax.fori_loop(0, NFULL // NBUF, gb, 0)
        for kk in range(NFULL, NCHK):
            consume(kk, bufs[kk % NBUF], sems[kk % NBUF])
        plsc.subcore_barrier()
        pltpu.sync_copy(acc.at[pl.ds(base, STRIPE)],
                        out_ref.at[c, pl.ds(base, STRIPE)])

    if linear:
        return k(table, dst2, tok)
    return k(table, src2, dst2, tok)


def _gather_rows(table_a, table_b, idx_a2, idx_b2, batch):
    per_w = batch // NW
    nchk = per_w // CHK

    @functools.partial(
        pl.kernel,
        out_type=(jax.ShapeDtypeStruct((batch, D), jnp.float32),
                  jax.ShapeDtypeStruct((batch, D), jnp.float32)),
        mesh=_mesh(),
        compiler_params=_params,
        scratch_types=(
            [pltpu.VMEM((nchk, CHK), jnp.int32)] * 2
            + [pltpu.VMEM((CHK, D), jnp.float32)] * 4
            + [pltpu.SemaphoreType.DMA] * 4
        ),
    )
    def k(ta_ref, tb_ref, ia_ref, ib_ref, oa_ref, ob_ref, ia, ib,
          a0, a1, b0, b1, sa0, sa1, sb0, sb1):
        c = lax.axis_index("c")
        s = lax.axis_index("s")
        wid = c * NS + s
        pltpu.sync_copy(ia_ref.at[pl.ds(wid * nchk, nchk)], ia)
        pltpu.sync_copy(ib_ref.at[pl.ds(wid * nchk, nchk)], ib)
        abufs = (a0, a1)
        bbufs = (b0, b1)
        asems = (sa0, sa1)
        bsems = (sb0, sb1)
        pltpu.async_copy(ta_ref.at[ia.at[0]], a0, sa0)
        pltpu.async_copy(tb_ref.at[ib.at[0]], b0, sb0)
        pltpu.async_copy(ta_ref.at[ia.at[1]], a1, sa1)
        pltpu.async_copy(tb_ref.at[ib.at[1]], b1, sb1)

        def eb(g, carry):
            for j in range(2):
                kk = g * 2 + j
                off = wid * per_w + kk * CHK

                pltpu.make_async_copy(ta_ref.at[ia.at[kk]], abufs[j],
                                      asems[j]).wait()
                pltpu.sync_copy(abufs[j], oa_ref.at[pl.ds(off, CHK)])

                @pl.when(kk + 2 < nchk)
                def _():
                    pltpu.async_copy(ta_ref.at[ia.at[kk + 2]], abufs[j],
                                     asems[j])
                pltpu.make_async_copy(tb_ref.at[ib.at[kk]], bbufs[j],
                                      bsems[j]).wait()
                pltpu.sync_copy(bbufs[j], ob_ref.at[pl.ds(off, CHK)])

                @pl.when(kk + 2 < nchk)
                def _():
                    pltpu.async_copy(tb_ref.at[ib.at[kk + 2]], bbufs[j],
                                     bsems[j])
            return carry
        lax.fori_loop(0, nchk // 2, eb, 0)
    return k(table_a, table_b, idx_a2, idx_b2)


def _mm_ee_body(a_ref, f_ref, s_ref, lo_ref, hi_ref):
    r = jnp.dot(a_ref[...], f_ref[...], preferred_element_type=jnp.float32)
    r = r * (0.7 * s_ref[...])
    lo_ref[...] = r[:, :HW]
    hi_ref[...] = r[:, HW:]


def _mm_ee(ef, F, s):
    blk = 8000
    return pl.pallas_call(
        _mm_ee_body,
        out_shape=(jax.ShapeDtypeStruct((E, HW), jnp.float32),
                   jax.ShapeDtypeStruct((E, HW), jnp.float32)),
        grid=(E // blk,),
        in_specs=[
            pl.BlockSpec((blk, D), lambda i: (i, 0)),
            pl.BlockSpec((D, D), lambda i: (0, 0)),
            pl.BlockSpec((blk, 1), lambda i: (i, 0)),
        ],
        out_specs=(pl.BlockSpec((blk, HW), lambda i: (i, 0)),
                   pl.BlockSpec((blk, HW), lambda i: (i, 0))),
    )(ef, F, s)


def _mm_if_body(a_ref, f_ref, lo_ref, hi_ref):
    r = jnp.dot(a_ref[...], f_ref[...], preferred_element_type=jnp.float32)
    r = r * 0.8
    lo_ref[...] = r[:, :HW]
    hi_ref[...] = r[:, HW:]


def _mm_if(ifeat_pad, F):
    blk = 1568
    return pl.pallas_call(
        _mm_if_body,
        out_shape=(jax.ShapeDtypeStruct((NPAD, HW), jnp.float32),
                   jax.ShapeDtypeStruct((NPAD, HW), jnp.float32)),
        grid=(NPAD // blk,),
        in_specs=[
            pl.BlockSpec((blk, D), lambda i: (i, 0)),
            pl.BlockSpec((D, D), lambda i: (0, 0)),
        ],
        out_specs=(pl.BlockSpec((blk, HW), lambda i: (i, 0)),
                   pl.BlockSpec((blk, HW), lambda i: (i, 0))),
    )(ifeat_pad, F)


_UBLK = 1568


def _nspec():
    return pl.BlockSpec((_UBLK, HW), lambda i: (i, 0))


def _pspec():
    return pl.BlockSpec((NC, _UBLK, HW), lambda i: (0, i, 0))


def _init_body(cil, cih, cul, cuh, gu, gi, dv,
               Cil, Cih, Cul, Cuh, yul, yuh, yil, yih):
    Cil[...] = cil[0] + cil[1]
    Cih[...] = cih[0] + cih[1]
    Cul[...] = cul[0] + cul[1]
    Cuh[...] = cuh[0] + cuh[1]
    sc = 0.3 * dv[...]
    yul[...] = sc * gu[:, :HW]
    yuh[...] = sc * gu[:, HW:]
    yil[...] = 0.2 * gi[:, :HW]
    yih[...] = 0.2 * gi[:, HW:]


def _init_pass(ci_lo, ci_hi, cu_lo, cu_hi, gu_pad, gi_pad, dinv2):
    outs = tuple(jax.ShapeDtypeStruct((NPAD, HW), jnp.float32) for _ in range(8))
    return pl.pallas_call(
        _init_body,
        out_shape=outs,
        grid=(NPAD // _UBLK,),
        in_specs=[
            _pspec(), _pspec(), _pspec(), _pspec(),
            pl.BlockSpec((_UBLK, D), lambda i: (i, 0)),
            pl.BlockSpec((_UBLK, D), lambda i: (i, 0)),
            pl.BlockSpec((_UBLK, 1), lambda i: (i, 0)),
        ],
        out_specs=tuple(_nspec() for _ in range(8)),
    )(ci_lo, ci_hi, cu_lo, cu_hi, gu_pad, gi_pad, dinv2)


def _make_update_body(alpha, want_y):
    def body(pil, pih, pul, puh, Cil, Cih, Cul, Cuh, dv,
             aul_i, auh_i, ail_i, aih_i, *outs):
        xil = Cil[...] + pil[0] + pil[1]
        xih = Cih[...] + pih[0] + pih[1]
        xul = Cul[...] + pul[0] + pul[1]
        xuh = Cuh[...] + puh[0] + puh[1]
        if want_y:
            (aul, auh, ail, aih, yul, yuh, yil, yih) = outs
            sc = 0.3 * dv[...]
            yul[...] = sc * xul
            yuh[...] = sc * xuh
            yil[...] = 0.2 * xil
            yih[...] = 0.2 * xih
        else:
            (aul, auh, ail, aih) = outs
        aul[...] = aul_i[...] + xul * alpha
        auh[...] = auh_i[...] + xuh * alpha
        ail[...] = ail_i[...] + xil * alpha
        aih[...] = aih_i[...] + xih * alpha
    return body


def _update_pass(pi_lo, pi_hi, pu_lo, pu_hi, Cil, Cih, Cul, Cuh, dinv2,
                 aul, auh, ail, aih, alpha, want_y):
    n_out = 8 if want_y else 4
    outs = tuple(jax.ShapeDtypeStruct((NPAD, HW), jnp.float32)
                 for _ in range(n_out))
    return pl.pallas_call(
        _make_update_body(alpha, want_y),
        out_shape=outs,
        grid=(NPAD // _UBLK,),
        in_specs=[
            _pspec(), _pspec(), _pspec(), _pspec(),
            _nspec(), _nspec(), _nspec(), _nspec(),
            pl.BlockSpec((_UBLK, 1), lambda i: (i, 0)),
            _nspec(), _nspec(), _nspec(), _nspec(),
        ],
        out_specs=tuple(_nspec() for _ in range(n_out)),
    )(pi_lo, pi_hi, pu_lo, pu_hi, Cil, Cih, Cul, Cuh, dinv2,
      aul, auh, ail, aih)


def _dot_body(a_ref, b_ref, o_ref):
    o_ref[...] = jnp.sum(a_ref[...] * b_ref[...], axis=1, keepdims=True)


def _batched_dot(a, b):
    B, Dd = a.shape
    blk = 1024
    return pl.pallas_call(
        _dot_body,
        out_shape=jax.ShapeDtypeStruct((B, 1), jnp.float32),
        grid=(B // blk,),
        in_specs=[
            pl.BlockSpec((blk, Dd), lambda i: (i, 0)),
            pl.BlockSpec((blk, Dd), lambda i: (i, 0)),
        ],
        out_specs=pl.BlockSpec((blk, 1), lambda i: (i, 0)),
    )(a, b)


def _pad_rows(x, n):
    return jnp.zeros((n, x.shape[1]), x.dtype).at[: x.shape[0]].set(x)


def kernel(Gu, Gi, F, edge_features, item_features, edge_index, user_idx, item_idx):
    u = edge_index[0, :E]
    items = edge_index[1, :E] - NU

    u_src = jnp.full((EPAD,), NU, jnp.int32).at[:E].set(u).reshape(-1, CHK)
    it_src = jnp.full((EPAD,), NI, jnp.int32).at[:E].set(items).reshape(-1, CHK)
    u_dst = jnp.full((EPAD,), TRASH, jnp.int32).at[:E].set(u).reshape(-1, CHK)
    it_dst = jnp.full((EPAD,), TRASH, jnp.int32).at[:E].set(items).reshape(-1, CHK)

    degp = _deg_pass(u_dst)
    deg = degp[0, :, 0] + degp[1, :, 0]
    dinv_pad = jnp.where(deg > 0, 1.0 / deg, 0.0)
    dinv2 = dinv_pad[:, None]

    s = _gather_scalar(dinv_pad, u_src)
    ee_lo, ee_hi = _mm_ee(edge_features, F, s[:E, None])
    if_lo, if_hi = _mm_if(_pad_rows(item_features, NPAD), F)

    ci_lo = _edge_pass(ee_lo, None, it_dst, s[:16], linear=True, src_limit=E)
    ci_hi = _edge_pass(ee_hi, None, it_dst, ci_lo[0, 0, :16], linear=True,
                       src_limit=E)
    cu_lo = _edge_pass(if_lo, it_src, u_dst, ci_hi[0, 0, :16])
    cu_hi = _edge_pass(if_hi, it_src, u_dst, cu_lo[0, 0, :16])

    gu_pad = _pad_rows(Gu, NPAD)
    gi_pad = _pad_rows(Gi, NPAD)
    (Cil, Cih, Cul, Cuh, yul, yuh, yil, yih) = _init_pass(
        ci_lo, ci_hi, cu_lo, cu_hi, gu_pad, gi_pad, dinv2)
    aul, auh = gu_pad[:, :HW], gu_pad[:, HW:]
    ail, aih = gi_pad[:, :HW], gi_pad[:, HW:]

    for layer in range(3):
        pi_lo = _edge_pass(yul, u_src, it_dst, yih[0, :16])
        pi_hi = _edge_pass(yuh, u_src, it_dst, pi_lo[0, 0, :16])
        pu_lo = _edge_pass(yil, it_src, u_dst, pi_hi[0, 0, :16])
        pu_hi = _edge_pass(yih, it_src, u_dst, pu_lo[0, 0, :16])
        alpha = 1.0 / (layer + 2)
        want_y = layer < 2
        res = _update_pass(pi_lo, pi_hi, pu_lo, pu_hi, Cil, Cih, Cul, Cuh,
                           dinv2, aul, auh, ail, aih, alpha, want_y)
        if want_y:
            (aul, auh, ail, aih, yul, yuh, yil, yih) = res
        else:
            (aul, auh, ail, aih) = res

    au = jnp.concatenate([aul, auh], axis=1)
    ai = jnp.concatenate([ail, aih], axis=1)
    ui2 = user_idx.astype(jnp.int32).reshape(-1, CHK)
    ii2 = item_idx.astype(jnp.int32).reshape(-1, CHK)
    ga, gb = _gather_rows(au, ai, ui2, ii2, user_idx.shape[0])
    return _batched_dot(ga, gb)[:, 0]
